# baseline (speedup 1.0000x reference)
import functools

import jax
import jax.numpy as jnp
from jax import lax
from jax.experimental import pallas as pl
from jax.experimental.pallas import tpu as pltpu
from jax.experimental.pallas import tpu_sc as plsc

F32 = jnp.float32
BF16 = jnp.bfloat16
_LOG2 = 0.6931471805599453

_NC = 2
_NS = 16
_NW = _NC * _NS

_TE = 4000
_TN = 2000


def _split(x):
    hi = x.astype(BF16)
    lo = (x - hi.astype(F32)).astype(BF16)
    return hi, lo


def _dot3(a, b, dims):
    ah, al = _split(a)
    bh, bl = _split(b)
    d = lambda p, q: lax.dot_general(p, q, (dims, ((), ())),
                                     preferred_element_type=F32)
    return d(ah, bh) + d(ah, bl) + d(al, bh)


def _mm(a, b):
    return _dot3(a, b, ((1,), (0,)))


def _gmm(oh, t):
    ohh = oh.astype(BF16)
    th, tl = _split(t)
    d = lambda p, q: lax.dot_general(p, q, (((1,), (0,)), ((), ())),
                                     preferred_element_type=F32)
    return d(ohh, th) + d(ohh, tl)


def _segsum(oh, x):
    ohh = oh.astype(BF16)
    xh, xl = _split(x)
    d = lambda p, q: lax.dot_general(p, q, (((0,), (0,)), ((), ())),
                                     preferred_element_type=F32)
    return d(ohh, xh) + d(ohh, xl)


def _sp2(x):
    return jnp.maximum(x, 0.0) + jnp.log1p(jnp.exp(-jnp.abs(x))) - _LOG2


def _elu(x):
    return jnp.where(x > 0, x, jnp.exp(jnp.minimum(x, 0.0)) - 1.0)


def _onehot(ids_col, n):
    return (ids_col == lax.broadcasted_iota(jnp.int32, (1, n), 1)).astype(F32)


def _onehot_bounds(cnt_row, T, G):
    tri = (lax.broadcasted_iota(jnp.int32, (G, G), 0)
           <= lax.broadcasted_iota(jnp.int32, (G, G), 1)).astype(F32)
    ic = _gmm(cnt_row, tri)
    lo = ic - cnt_row
    r = (pl.program_id(0) * T
         + lax.broadcasted_iota(jnp.int32, (T, 1), 0)).astype(F32)
    return ((r >= lo) & (r < ic)).astype(F32)


def _acc(ref, val, first):
    tot = jnp.where(first, val, ref[...] + val)
    ref[...] = tot
    return tot


def _full(a):
    return pl.BlockSpec(a.shape, lambda i: tuple(0 for _ in a.shape))


def _cfix(shape):
    return pl.BlockSpec(shape, lambda i: tuple(0 for _ in shape))


def _node_encoder(nf3, embp, w1, b1, w2, b2, sa, sw1, sb1, sw2, sb2):
    grid, T, _ = nf3.shape
    N = grid * T
    G = sa.shape[0]

    def body(nf_ref, emb_ref, w1_ref, b1_ref, w2_ref, b2_ref,
             sa_ref, sw1_ref, sb1_ref, sw2_ref, sb2_ref, out_ref, u_ref):
        oh = _onehot(nf_ref[0], 128)
        vemb = _gmm(oh, emb_ref[...])
        h = _sp2(_mm(vemb, w1_ref[...]) + b1_ref[...])
        out_ref[...] = _sp2(_mm(h, w2_ref[...]) + b2_ref[...])

        @pl.when(pl.program_id(0) == 0)
        def _():
            hs = _sp2(_mm(sa_ref[...], sw1_ref[...]) + sb1_ref[...])
            u_ref[...] = _sp2(_mm(hs, sw2_ref[...]) + sb2_ref[...])

    return pl.pallas_call(
        body,
        grid=(grid,),
        in_specs=[
            pl.BlockSpec((1, T, 1), lambda i: (i, 0, 0)),
            _full(embp), _full(w1), _full(b1), _full(w2), _full(b2),
            _full(sa), _full(sw1), _full(sb1), _full(sw2), _full(sb2),
        ],
        out_specs=[pl.BlockSpec((T, 32), lambda i: (i, 0)),
                   _cfix((G, 32))],
        out_shape=[jax.ShapeDtypeStruct((N, 32), F32),
                   jax.ShapeDtypeStruct((G, 32), F32)],
    )(nf3, embp, w1, b1, w2, b2, sa, sw1, sb1, sw2, sb2)


def _edge_block(vs, vd, e_or_bd, u, w1, b1, w2, b2, w3, b3,
                gi3=None, enc=None, ecnt=None):
    E = vs.shape[0]
    G = u.shape[0]
    T = _TE
    grid = E // T
    first_blk = enc is not None

    def body(*refs):
        if first_blk:
            (vs_ref, vd_ref, e_ref, gi_ref, u_ref,
             cw1_ref, cb1_ref, cw2_ref, cb2_ref,
             w1_ref, b1_ref, w2_ref, b2_ref, w3_ref, b3_ref,
             em_ref, en_ref, q0_ref, cc_ref, cr_ref) = refs
            d = e_ref[...]
            cc = lax.broadcasted_iota(jnp.int32, (1, 128), 1).astype(F32) * (5.0 / 99.0)
            rbf = jnp.exp(-((d - cc) ** 2) * 4.0)
            eh = _sp2(_mm(rbf, cw1_ref[...]) + cb1_ref[...])
            e = _sp2(_mm(eh, cw2_ref[...]) + cb2_ref[...])
            oh = _onehot(gi_ref[0], G)
        else:
            (vs_ref, vd_ref, e_ref, ec_ref, u_ref,
             w1_ref, b1_ref, w2_ref, b2_ref, w3_ref, b3_ref,
             em_ref, en_ref) = refs
            e = e_ref[...]
            oh = _onehot_bounds(ec_ref[...], T, G)
        ue = _gmm(oh, u_ref[...])
        x = jnp.concatenate([vs_ref[...], vd_ref[...], e, ue], axis=1)
        h = _sp2(_mm(x, w1_ref[...]) + b1_ref[...])
        h = _sp2(_mm(h, w2_ref[...]) + b2_ref[...])
        em = _sp2(_mm(h, w3_ref[...]) + b3_ref[...])
        en = em + e
        em_ref[...] = em
        en_ref[...] = en
        if first_blk:
            first = pl.program_id(0) == 0
            both = _segsum(oh, jnp.concatenate(
                [e, jnp.ones((T, 1), F32)], axis=1))
            _acc(q0_ref, both[:, 0:32], first)
            _acc(cc_ref, both[:, 32:33], first)
            _acc(cr_ref, jnp.sum(oh, axis=0, keepdims=True), first)

    row = lambda w: pl.BlockSpec((T, w), lambda i: (i, 0))
    if first_blk:
        in_specs = [row(32), row(32), row(1),
                    pl.BlockSpec((1, T, 1), lambda i: (i, 0, 0)), _full(u),
                    _full(enc[0]), _full(enc[1]), _full(enc[2]), _full(enc[3]),
                    _full(w1), _full(b1), _full(w2), _full(b2), _full(w3),
                    _full(b3)]
        args = (vs, vd, e_or_bd, gi3, u, *enc, w1, b1, w2, b2, w3, b3)
        out_specs = [row(32), row(32), _cfix((G, 32)),
                     _cfix((G, 1)), _cfix((1, G))]
        out_shape = [jax.ShapeDtypeStruct((E, 32), F32),
                     jax.ShapeDtypeStruct((E, 32), F32),
                     jax.ShapeDtypeStruct((G, 32), F32),
                     jax.ShapeDtypeStruct((G, 1), F32),
                     jax.ShapeDtypeStruct((1, G), F32)]
    else:
        in_specs = [row(32), row(32), row(32), _full(ecnt), _full(u),
                    _full(w1), _full(b1), _full(w2), _full(b2), _full(w3),
                    _full(b3)]
        args = (vs, vd, e_or_bd, ecnt, u, w1, b1, w2, b2, w3, b3)
        out_specs = [row(32), row(32)]
        out_shape = [jax.ShapeDtypeStruct((E, 32), F32),
                     jax.ShapeDtypeStruct((E, 32), F32)]
    return pl.pallas_call(
        body, grid=(grid,), in_specs=in_specs, out_specs=out_specs,
        out_shape=out_shape,
    )(*args)


def _node_block(v0, a0, a1, c0, c1, u, w1, b1, w2, b2, w3, b3,
                sg0, sg1, qe_prev, ce, uw1, ub1, uw2, ub2, uw3, ub3,
                gi3=None, vcnt_row=None, vcnt_col=None):
    N = v0.shape[0]
    G = u.shape[0]
    T = _TN
    grid = N // T
    first_blk = gi3 is not None

    def body(*refs):
        if first_blk:
            (v_ref, a0_ref, a1_ref, c0_ref, c1_ref, gi_ref, u_ref,
             w1_ref, b1_ref, w2_ref, b2_ref, w3_ref, b3_ref,
             sg0_ref, sg1_ref, qe_ref, ce_ref,
             uw1_ref, ub1_ref, uw2_ref, ub2_ref, uw3_ref, ub3_ref,
             vn_ref, sq_ref, un_ref, qe_out, cc_ref, cr_ref, sp_ref) = refs
            oh = _onehot(gi_ref[0], G)
        else:
            (v_ref, a0_ref, a1_ref, c0_ref, c1_ref, vc_ref, vcc_ref, u_ref,
             w1_ref, b1_ref, w2_ref, b2_ref, w3_ref, b3_ref,
             sg0_ref, sg1_ref, qe_ref, ce_ref,
             uw1_ref, ub1_ref, uw2_ref, ub2_ref, uw3_ref, ub3_ref,
             vn_ref, sq_ref, un_ref, qe_out, sp_ref) = refs
            oh = _onehot_bounds(vc_ref[...], T, G)
        cnt = jnp.maximum(c0_ref[:, 0:1] + c1_ref[:, 0:1], 1.0)
        em = (a0_ref[...] + a1_ref[...]) / cnt
        uv = _gmm(oh, u_ref[...])
        x = jnp.concatenate([v_ref[...], em, uv], axis=1)
        h = _sp2(_mm(x, w1_ref[...]) + b1_ref[...])
        h = _sp2(_mm(h, w2_ref[...]) + b2_ref[...])
        vm = _sp2(_mm(h, w3_ref[...]) + b3_ref[...])
        vn = vm + v_ref[...]
        vn_ref[...] = vn
        first = pl.program_id(0) == 0
        if first_blk:
            both = _segsum(oh, jnp.concatenate(
                [vm, vn, jnp.ones((T, 1), F32)], axis=1))
            sp_tot = _acc(sp_ref, both[:, 0:32], first)
            _acc(sq_ref, both[:, 32:64], first)
            cv_tot = _acc(cc_ref, both[:, 64:65], first)
            _acc(cr_ref, jnp.sum(oh, axis=0, keepdims=True), first)
        else:
            both = _segsum(oh, jnp.concatenate([vm, vn], axis=1))
            sp_tot = _acc(sp_ref, both[:, 0:32], first)
            _acc(sq_ref, both[:, 32:64], first)
            cv_tot = vcc_ref[...]

        @pl.when(pl.program_id(0) == grid - 1)
        def _():
            se = sg0_ref[...] + sg1_ref[...]
            qe_out[...] = qe_ref[...] + se
            me = se / jnp.maximum(ce_ref[...], 1.0)
            mv = sp_tot / jnp.maximum(cv_tot, 1.0)
            xs = jnp.concatenate([u_ref[...], me, mv], axis=1)
            hs = _sp2(_mm(xs, uw1_ref[...]) + ub1_ref[...])
            hs = _sp2(_mm(hs, uw2_ref[...]) + ub2_ref[...])
            un_ref[...] = _sp2(_mm(hs, uw3_ref[...]) + ub3_ref[...]) + u_ref[...]

    row = lambda w: pl.BlockSpec((T, w), lambda i: (i, 0))
    w_specs = [_full(x) for x in
               (u, w1, b1, w2, b2, w3, b3, sg0, sg1, qe_prev, ce,
                uw1, ub1, uw2, ub2, uw3, ub3)]
    if first_blk:
        in_specs = ([row(32)] * 5
                    + [pl.BlockSpec((1, T, 1), lambda i: (i, 0, 0))] + w_specs)
        args = (v0, a0, a1, c0, c1, gi3, u, w1, b1, w2, b2, w3, b3,
                sg0, sg1, qe_prev, ce, uw1, ub1, uw2, ub2, uw3, ub3)
        out_specs = [row(32), _cfix((G, 32)), _cfix((G, 32)), _cfix((G, 32)),
                     _cfix((G, 1)), _cfix((1, G)), _cfix((G, 32))]
        out_shape = [jax.ShapeDtypeStruct((N, 32), F32),
                     jax.ShapeDtypeStruct((G, 32), F32),
                     jax.ShapeDtypeStruct((G, 32), F32),
                     jax.ShapeDtypeStruct((G, 32), F32),
                     jax.ShapeDtypeStruct((G, 1), F32),
                     jax.ShapeDtypeStruct((1, G), F32),
                     jax.ShapeDtypeStruct((G, 32), F32)]
    else:
        in_specs = ([row(32)] * 5 + [_full(vcnt_row), _full(vcnt_col)]
                    + w_specs)
        args = (v0, a0, a1, c0, c1, vcnt_row, vcnt_col, u,
                w1, b1, w2, b2, w3, b3, sg0, sg1, qe_prev, ce,
                uw1, ub1, uw2, ub2, uw3, ub3)
        out_specs = [row(32), _cfix((G, 32)), _cfix((G, 32)), _cfix((G, 32)),
                     _cfix((G, 32))]
        out_shape = [jax.ShapeDtypeStruct((N, 32), F32),
                     jax.ShapeDtypeStruct((G, 32), F32),
                     jax.ShapeDtypeStruct((G, 32), F32),
                     jax.ShapeDtypeStruct((G, 32), F32),
                     jax.ShapeDtypeStruct((G, 32), F32)]
    return pl.pallas_call(
        body, grid=(grid,), in_specs=in_specs, out_specs=out_specs,
        out_shape=out_shape,
    )(*args)


def _s2s_pass_a(x, h_or_lstm, cnt_row):
    R = x.shape[0]
    G = cnt_row.shape[1]
    T = _TE if R % _TE == 0 else _TN
    grid = R // T
    with_lstm = isinstance(h_or_lstm, tuple)

    def lstm_half(r1, w_ref, b_ref, d):
        g = _mm(r1, w_ref[d:, :]) + b_ref[...]
        i = jax.nn.sigmoid(g[:, 0:d])
        gg = jnp.tanh(g[:, 2 * d:3 * d])
        o = jax.nn.sigmoid(g[:, 3 * d:4 * d])
        return o * jnp.tanh(i * gg)

    def body(*refs):
        if with_lstm:
            (x_ref, ec_ref, sv_ref, cv_ref, se_ref, ce_ref,
             wn_ref, bn_ref, we_ref, be_ref,
             m_ref, hn_ref, he_ref, h_scr) = refs

            @pl.when(pl.program_id(0) == 0)
            def _():
                rn = sv_ref[...] / jnp.maximum(cv_ref[...], 1.0)
                re = se_ref[...] / jnp.maximum(ce_ref[...], 1.0)
                hn = lstm_half(rn, wn_ref, bn_ref, 32)
                hn_ref[...] = hn
                he_ref[...] = lstm_half(re, we_ref, be_ref, 32)
                h_scr[...] = hn

            h = h_scr[...]
        else:
            x_ref, ec_ref, h_ref, m_ref = refs
            h = h_ref[...]
        oh = _onehot_bounds(ec_ref[...], T, G)
        hseg = _gmm(oh, h)
        s = jnp.sum(x_ref[...] * hseg, axis=1, keepdims=True)
        mp = jnp.max(jnp.where(oh > 0, s, -1e30), axis=0, keepdims=True)
        first = pl.program_id(0) == 0
        m_ref[...] = jnp.where(first, mp, jnp.maximum(m_ref[...], mp))

    row32 = pl.BlockSpec((T, 32), lambda i: (i, 0))
    if with_lstm:
        sv, cv, se, ce, wn, bn, we, be = h_or_lstm
        return pl.pallas_call(
            body, grid=(grid,),
            in_specs=[row32, _full(cnt_row), _full(sv), _full(cv), _full(se),
                      _full(ce), _full(wn), _full(bn), _full(we), _full(be)],
            out_specs=[_cfix((1, G)), _cfix((G, 32)), _cfix((G, 32))],
            out_shape=[jax.ShapeDtypeStruct((1, G), F32),
                       jax.ShapeDtypeStruct((G, 32), F32),
                       jax.ShapeDtypeStruct((G, 32), F32)],
            scratch_shapes=[pltpu.VMEM((G, 32), F32)],
        )(x, cnt_row, sv, cv, se, ce, wn, bn, we, be)
    h = h_or_lstm
    return pl.pallas_call(
        body, grid=(grid,),
        in_specs=[row32, _full(cnt_row), _full(h)],
        out_specs=_cfix((1, G)),
        out_shape=jax.ShapeDtypeStruct((1, G), F32),
    )(x, cnt_row, h)


def _s2s_pass_b(x, h, cnt_row, m, readout=None):
    R = x.shape[0]
    G = m.shape[1]
    T = _TE if R % _TE == 0 else _TN
    grid = R // T

    def body(*refs):
        if readout is None:
            x_ref, ec_ref, h_ref, m_ref, d_ref, n_ref = refs
        else:
            (x_ref, ec_ref, h_ref, m_ref,
             hn_ref, nn_ref, dnr, u_ref,
             ow1r, ob1r, ow2r, ob2r, opwr, opbr,
             latr, lw1r, lb1r, lw2r, lb2r, lw3r, lb3r,
             fprer, pw1r, pb1r, pw2r, pb2r, pw3r, pb3r,
             sgr, embr, woutr, boutr, out_ref, d_ref, n_ref) = refs
        oh = _onehot_bounds(ec_ref[...], T, G)
        hseg = _gmm(oh, h_ref[...])
        xv = x_ref[...]
        s = jnp.sum(xv * hseg, axis=1, keepdims=True)
        mseg = jnp.sum(oh * m_ref[...], axis=1, keepdims=True)
        a = jnp.exp(s - mseg)
        both = _segsum(oh, jnp.concatenate([a, a * xv], axis=1))
        first = pl.program_id(0) == 0
        de = _acc(d_ref, both[:, 0:1], first)
        ne = _acc(n_ref, both[:, 1:33], first)
        if readout is not None:
            @pl.when(pl.program_id(0) == grid - 1)
            def _():
                rn = nn_ref[...] / (dnr[...] + 1e-12)
                re = ne / (de + 1e-12)
                z = jnp.concatenate(
                    [hn_ref[...], rn, h_ref[...], re, u_ref[...]], axis=1)
                z = _sp2(_mm(z, ow1r[...]) + ob1r[...])
                z = _sp2(_mm(z, ow2r[...]) + ob2r[...])
                xa = _mm(z, opwr[...]) + opbr[...]
                xl = _elu(_mm(latr[...], lw1r[...]) + lb1r[...])
                xl = _elu(_mm(xl, lw2r[...]) + lb2r[...])
                xl = _elu(_mm(xl, lw3r[...]) + lb3r[...])
                xp = _elu(_mm(fprer[...], pw1r[...]) + pb1r[...])
                xp = _elu(_mm(xp, pw2r[...]) + pb2r[...])
                xp = _elu(_mm(xp, pw3r[...]) + pb3r[...])
                ohg = _onehot(sgr[0], 256)
                xs = _gmm(ohg, embr[...])
                fx = jnp.concatenate([xa, xl, xp, xs], axis=1)
                out_ref[...] = _mm(fx, woutr[...]) + boutr[...]

    if readout is None:
        return pl.pallas_call(
            body, grid=(grid,),
            in_specs=[pl.BlockSpec((T, 32), lambda i: (i, 0)),
                      _full(cnt_row), _full(h), _full(m)],
            out_specs=[_cfix((G, 1)), _cfix((G, 32))],
            out_shape=[jax.ShapeDtypeStruct((G, 1), F32),
                       jax.ShapeDtypeStruct((G, 32), F32)],
        )(x, cnt_row, h, m)
    ro = readout
    sg3 = ro[24]
    return pl.pallas_call(
        body, grid=(grid,),
        in_specs=([pl.BlockSpec((T, 32), lambda i: (i, 0)),
                   _full(cnt_row), _full(h), _full(m)]
                  + [_full(a) for a in ro[:24]]
                  + [pl.BlockSpec(sg3.shape, lambda i: (0, 0, 0))]
                  + [_full(a) for a in ro[25:]]),
        out_specs=[_cfix((G, 128)), _cfix((G, 1)), _cfix((G, 32))],
        out_shape=[jax.ShapeDtypeStruct((G, 128), F32),
                   jax.ShapeDtypeStruct((G, 1), F32),
                   jax.ShapeDtypeStruct((G, 32), F32)],
    )(x, cnt_row, h, m, *ro)[0]


def _sc_gather_pair(table, isrc, idst):
    E = isrc.shape[0]
    D = table.shape[1]
    epw = E // _NW
    ch = 400
    nch = epw // ch
    mesh = plsc.VectorSubcoreMesh(core_axis_name="c", subcore_axis_name="s")

    @functools.partial(
        pl.kernel,
        out_type=[
            jax.ShapeDtypeStruct((E, D), F32),
            jax.ShapeDtypeStruct((E, D), F32),
        ],
        mesh=mesh,
        compiler_params=pltpu.CompilerParams(use_tc_tiling_on_sc=False),
        scratch_types=[
            [pltpu.VMEM((ch,), jnp.int32)] * 2,
            [pltpu.VMEM((ch, D), F32)] * 2,
            [pltpu.VMEM((ch,), jnp.int32)] * 2,
            [pltpu.VMEM((ch, D), F32)] * 2,
            [pltpu.SemaphoreType.DMA] * 2,
            [pltpu.SemaphoreType.DMA] * 2,
        ],
    )
    def k(table_h, isrc_h, idst_h, osrc_h, odst_h, iv1, rv1, iv2, rv2, s1, s2):
        wid = lax.axis_index("s") * _NC + lax.axis_index("c")
        base = wid * epw

        def load_and_fire(c):
            b = c % 2
            off = base + c * ch
            pltpu.sync_copy(isrc_h.at[pl.ds(off, ch)], iv1[b])
            pltpu.sync_copy(idst_h.at[pl.ds(off, ch)], iv2[b])
            g1 = pltpu.async_copy(table_h.at[iv1[b]], rv1[b], s1[b])
            g2 = pltpu.async_copy(table_h.at[iv2[b]], rv2[b], s2[b])
            return g1, g2

        def drain(c, g1, g2):
            b = c % 2
            off = base + c * ch
            g1.wait()
            g2.wait()
            pltpu.sync_copy(rv1[b], osrc_h.at[pl.ds(off, ch)])
            pltpu.sync_copy(rv2[b], odst_h.at[pl.ds(off, ch)])

        pend = load_and_fire(0)
        for c in range(1, nch):
            nxt = load_and_fire(c)
            drain(c - 1, *pend)
            pend = nxt
        drain(nch - 1, *pend)

    return k(table, isrc, idst)


def _sc_scatter_add(data, idx, gidx, nrows, G, zeros):
    E = idx.shape[0]
    D = data.shape[1]
    epw = E // _NW
    ch = 2000
    nch = epw // ch
    stripe = nrows // _NS
    gstripe = G // _NS
    mesh = plsc.VectorSubcoreMesh(core_axis_name="c", subcore_axis_name="s")

    @functools.partial(
        pl.kernel,
        out_type=[jax.ShapeDtypeStruct((_NC, nrows, D), F32),
                  jax.ShapeDtypeStruct((_NC, G, D), F32)],
        mesh=mesh,
        compiler_params=pltpu.CompilerParams(use_tc_tiling_on_sc=False),
        scratch_types=[
            pltpu.VMEM((ch,), jnp.int32),
            pltpu.VMEM((ch,), jnp.int32),
            pltpu.VMEM((ch, D), F32),
            pltpu.VMEM_SHARED((nrows, D), F32),
            pltpu.VMEM_SHARED((G, D), F32),
        ],
    )
    def k(data_h, idx_h, gidx_h, zeros_h, out_h, gout_h, iv, gv, dv, acc, gacc):
        cid = lax.axis_index("c")
        sid = lax.axis_index("s")
        wid = sid * _NC + cid
        pltpu.sync_copy(zeros_h.at[pl.ds(sid * stripe, stripe)],
                        acc.at[pl.ds(sid * stripe, stripe)])
        pltpu.sync_copy(zeros_h.at[pl.ds(sid * gstripe, gstripe)],
                        gacc.at[pl.ds(sid * gstripe, gstripe)])
        plsc.subcore_barrier()
        base = wid * epw
        for c in range(nch):
            off = base + c * ch
            pltpu.sync_copy(idx_h.at[pl.ds(off, ch)], iv)
            pltpu.sync_copy(gidx_h.at[pl.ds(off, ch)], gv)
            pltpu.sync_copy(data_h.at[pl.ds(off, ch)], dv)
            pltpu.sync_copy(dv, acc.at[iv], add=True)
            pltpu.sync_copy(dv, gacc.at[gv], add=True)
        plsc.subcore_barrier()
        pltpu.sync_copy(acc.at[pl.ds(sid * stripe, stripe)],
                        out_h.at[cid, pl.ds(sid * stripe, stripe)])
        pltpu.sync_copy(gacc.at[pl.ds(sid * gstripe, gstripe)],
                        gout_h.at[cid, pl.ds(sid * gstripe, gstripe)])

    return k(data, idx, gidx, zeros)


def _sc_scatter_ones(idx, nrows, zeros, ones_ch):
    E = idx.shape[0]
    ch, D = ones_ch.shape
    epw = E // _NW
    nch = epw // ch
    stripe = nrows // _NS
    mesh = plsc.VectorSubcoreMesh(core_axis_name="c", subcore_axis_name="s")

    @functools.partial(
        pl.kernel,
        out_type=jax.ShapeDtypeStruct((_NC, nrows, D), F32),
        mesh=mesh,
        compiler_params=pltpu.CompilerParams(use_tc_tiling_on_sc=False),
        scratch_types=[
            pltpu.VMEM((ch,), jnp.int32),
            pltpu.VMEM((ch, D), F32),
            pltpu.VMEM_SHARED((nrows, D), F32),
        ],
    )
    def k(idx_h, zeros_h, ones_h, out_h, iv, dv, acc):
        cid = lax.axis_index("c")
        sid = lax.axis_index("s")
        wid = sid * _NC + cid
        pltpu.sync_copy(zeros_h.at[pl.ds(sid * stripe, stripe)],
                        acc.at[pl.ds(sid * stripe, stripe)])
        pltpu.sync_copy(ones_h, dv)
        plsc.subcore_barrier()
        base = wid * epw
        for c in range(nch):
            off = base + c * ch
            pltpu.sync_copy(idx_h.at[pl.ds(off, ch)], iv)
            pltpu.sync_copy(dv, acc.at[iv], add=True)
        plsc.subcore_barrier()
        pltpu.sync_copy(acc.at[pl.ds(sid * stripe, stripe)],
                        out_h.at[cid, pl.ds(sid * stripe, stripe)])

    return k(idx, zeros, ones_ch)


def kernel(edge_index, bond_dist, node_feat, state_attr, node_graph_idx,
           edge_graph_idx, sg, lattice, fpretrain, params):
    E = bond_dist.shape[0]
    N = node_feat.shape[0]
    G = state_attr.shape[0]

    src = edge_index[0].astype(jnp.int32)
    dst = edge_index[1].astype(jnp.int32)
    gi_e3 = edge_graph_idx.astype(jnp.int32).reshape(E // _TE, _TE, 1)
    gi_n3 = node_graph_idx.astype(jnp.int32).reshape(N // _TN, _TN, 1)
    nf3 = node_feat.astype(jnp.int32).reshape(N // _TN, _TN, 1)
    sg3 = sg.astype(jnp.int32).reshape(1, G, 1)
    bd = bond_dist.reshape(E, 1)

    p = params
    rb = lambda b: b.reshape(1, -1)

    (ew1, eb1), (ew2, eb2) = p['edge_enc']
    enc = (jnp.pad(ew1, ((0, 28), (0, 0))), rb(eb1), ew2, rb(eb2))

    (nw1, nb1), (nw2, nb2) = p['node_enc']
    (sw1, sb1), (sw2, sb2) = p['state_enc']
    embp = jnp.pad(p['emb_atom'], ((0, 33), (0, 0)))
    v, u = _node_encoder(nf3, embp, nw1, rb(nb1), nw2, rb(nb2),
                         state_attr, sw1, rb(sb1), sw2, rb(sb2))

    zeros_n = jnp.zeros((N, 32), F32)
    counts = _sc_scatter_ones(dst, N, zeros_n, jnp.ones((2000, 32), F32))
    c0, c1 = counts[0], counts[1]
    gie = edge_graph_idx.astype(jnp.int32)

    e_in = bd
    ecnt_col = ecnt_row = vcnt_row = vcnt_col = None
    qe = None
    seg_v_post = None
    for bi, blk in enumerate(p['blocks']):
        (bw1, bb1), (bw2, bb2), (bw3, bb3) = blk['edge']
        vs, vd = _sc_gather_pair(v, src, dst)
        if bi == 0:
            em, en, q0, ecnt_col, ecnt_row = _edge_block(
                vs, vd, e_in, u, bw1, rb(bb1), bw2, rb(bb2), bw3, rb(bb3),
                gi3=gi_e3, enc=enc)
            qe = q0
        else:
            em, en = _edge_block(
                vs, vd, e_in, u, bw1, rb(bb1), bw2, rb(bb2), bw3, rb(bb3),
                ecnt=ecnt_row)
        aggs, gsums = _sc_scatter_add(em, dst, gie, N, G, zeros_n)
        (vw1, vb1), (vw2, vb2), (vw3, vb3) = blk['node']
        (uw1, ub1), (uw2, ub2), (uw3, ub3) = blk['state']
        if bi == 0:
            vn, sv_post, un, qe, vcnt_col, vcnt_row, _sv_pre = _node_block(
                v, aggs[0], aggs[1], c0, c1, u,
                vw1, rb(vb1), vw2, rb(vb2), vw3, rb(vb3),
                gsums[0], gsums[1], qe, ecnt_col,
                uw1, rb(ub1), uw2, rb(ub2), uw3, rb(ub3),
                gi3=gi_n3)
        else:
            vn, sv_post, un, qe, _sv_pre = _node_block(
                v, aggs[0], aggs[1], c0, c1, u,
                vw1, rb(vb1), vw2, rb(vb2), vw3, rb(vb3),
                gsums[0], gsums[1], qe, ecnt_col,
                uw1, rb(ub1), uw2, rb(ub2), uw3, rb(ub3),
                vcnt_row=vcnt_row, vcnt_col=vcnt_col)
        e_in, v, u = en, vn, un
        seg_v_post = sv_post

    mn, hn, he = _s2s_pass_a(
        v, (seg_v_post, vcnt_col, qe, ecnt_col,
            p['s2s_node']['Wih'], rb(p['s2s_node']['b']),
            p['s2s_edge']['Wih'], rb(p['s2s_edge']['b'])), vcnt_row)
    me_ = _s2s_pass_a(e_in, he, ecnt_row)
    den_n, num_n = _s2s_pass_b(v, hn, vcnt_row, mn)

    (ow1, ob1), (ow2, ob2) = p['out_mlp']
    opw, opb = p['out_proj']
    (lw1, lb1), (lw2, lb2), (lw3, lb3) = p['emb_lattice']
    (pw1, pb1), (pw2, pb2), (pw3, pb3) = p['emb_pretrain']
    embsg = jnp.pad(p['emb_sg'], ((0, 26), (0, 0)))
    wout, bout = p['output_layer']

    return _s2s_pass_b(
        e_in, he, ecnt_row, me_,
        readout=(hn, num_n, den_n, u,
                 ow1, rb(ob1), ow2, rb(ob2), opw, rb(opb),
                 lattice, lw1, rb(lb1), lw2, rb(lb2), lw3, rb(lb3),
                 fpretrain, pw1, rb(pb1), pw2, rb(pb2), pw3, rb(pb3),
                 sg3, embsg, wout, rb(bout)))


# speedup vs baseline: 1.0403x; 1.0403x over previous
import functools

import jax
import jax.numpy as jnp
from jax import lax
from jax.experimental import pallas as pl
from jax.experimental.pallas import tpu as pltpu
from jax.experimental.pallas import tpu_sc as plsc

F32 = jnp.float32
BF16 = jnp.bfloat16
_LOG2 = 0.6931471805599453

_NC = 2
_NS = 16
_NW = _NC * _NS

_TE = 4000
_TN = 2000


def _split(x):
    hi = x.astype(BF16)
    lo = (x - hi.astype(F32)).astype(BF16)
    return hi, lo


def _dot3(a, b, dims):
    ah, al = _split(a)
    bh, bl = _split(b)
    d = lambda p, q: lax.dot_general(p, q, (dims, ((), ())),
                                     preferred_element_type=F32)
    return d(ah, bh) + d(ah, bl) + d(al, bh)


def _mm(a, b):
    return _dot3(a, b, ((1,), (0,)))


def _gmm(oh, t):
    ohh = oh.astype(BF16)
    th, tl = _split(t)
    d = lambda p, q: lax.dot_general(p, q, (((1,), (0,)), ((), ())),
                                     preferred_element_type=F32)
    return d(ohh, th) + d(ohh, tl)


def _segsum(oh, x):
    ohh = oh.astype(BF16)
    xh, xl = _split(x)
    d = lambda p, q: lax.dot_general(p, q, (((0,), (0,)), ((), ())),
                                     preferred_element_type=F32)
    return d(ohh, xh) + d(ohh, xl)


def _sp2(x):
    return jnp.maximum(x, 0.0) + jnp.log1p(jnp.exp(-jnp.abs(x))) - _LOG2


def _elu(x):
    return jnp.where(x > 0, x, jnp.exp(jnp.minimum(x, 0.0)) - 1.0)


def _onehot(ids_col, n):
    return (ids_col == lax.broadcasted_iota(jnp.int32, (1, n), 1)).astype(F32)


def _onehot_bounds(cnt_row, T, G):
    tri = (lax.broadcasted_iota(jnp.int32, (G, G), 0)
           <= lax.broadcasted_iota(jnp.int32, (G, G), 1)).astype(F32)
    ic = _gmm(cnt_row, tri)
    lo = ic - cnt_row
    r = (pl.program_id(0) * T
         + lax.broadcasted_iota(jnp.int32, (T, 1), 0)).astype(F32)
    return ((r >= lo) & (r < ic)).astype(F32)


def _acc(ref, val, first):
    tot = jnp.where(first, val, ref[...] + val)
    ref[...] = tot
    return tot


def _full(a):
    return pl.BlockSpec(a.shape, lambda i: tuple(0 for _ in a.shape))


def _cfix(shape):
    return pl.BlockSpec(shape, lambda i: tuple(0 for _ in shape))


def _node_encoder(nf3, embp, w1, b1, w2, b2, sa, sw1, sb1, sw2, sb2):
    grid, T, _ = nf3.shape
    N = grid * T
    G = sa.shape[0]

    def body(nf_ref, emb_ref, w1_ref, b1_ref, w2_ref, b2_ref,
             sa_ref, sw1_ref, sb1_ref, sw2_ref, sb2_ref, out_ref, u_ref):
        oh = _onehot(nf_ref[0], 128)
        vemb = _gmm(oh, emb_ref[...])
        h = _sp2(_mm(vemb, w1_ref[...]) + b1_ref[...])
        out_ref[...] = _sp2(_mm(h, w2_ref[...]) + b2_ref[...])

        @pl.when(pl.program_id(0) == 0)
        def _():
            hs = _sp2(_mm(sa_ref[...], sw1_ref[...]) + sb1_ref[...])
            u_ref[...] = _sp2(_mm(hs, sw2_ref[...]) + sb2_ref[...])

    return pl.pallas_call(
        body,
        grid=(grid,),
        in_specs=[
            pl.BlockSpec((1, T, 1), lambda i: (i, 0, 0)),
            _full(embp), _full(w1), _full(b1), _full(w2), _full(b2),
            _full(sa), _full(sw1), _full(sb1), _full(sw2), _full(sb2),
        ],
        out_specs=[pl.BlockSpec((T, 32), lambda i: (i, 0)),
                   _cfix((G, 32))],
        out_shape=[jax.ShapeDtypeStruct((N, 32), F32),
                   jax.ShapeDtypeStruct((G, 32), F32)],
    )(nf3, embp, w1, b1, w2, b2, sa, sw1, sb1, sw2, sb2)


def _edge_block(vs, vd, e_or_bd, u, w1, b1, w2, b2, w3, b3,
                enc=None, ecnt=None):
    E = vs.shape[0]
    G = u.shape[0]
    T = _TE
    grid = E // T
    first_blk = enc is not None

    def body(*refs):
        if first_blk:
            (vs_ref, vd_ref, e_ref, ec_ref, u_ref,
             cw1_ref, cb1_ref, cw2_ref, cb2_ref,
             w1_ref, b1_ref, w2_ref, b2_ref, w3_ref, b3_ref,
             em_ref, en_ref, q0_ref) = refs
            d = e_ref[...]
            cc = lax.broadcasted_iota(jnp.int32, (1, 128), 1).astype(F32) * (5.0 / 99.0)
            rbf = jnp.exp(-((d - cc) ** 2) * 4.0)
            eh = _sp2(_mm(rbf, cw1_ref[...]) + cb1_ref[...])
            e = _sp2(_mm(eh, cw2_ref[...]) + cb2_ref[...])
        else:
            (vs_ref, vd_ref, e_ref, ec_ref, u_ref,
             w1_ref, b1_ref, w2_ref, b2_ref, w3_ref, b3_ref,
             em_ref, en_ref) = refs
            e = e_ref[...]
        oh = _onehot_bounds(ec_ref[...], T, G)
        ue = _gmm(oh, u_ref[...])
        x = jnp.concatenate([vs_ref[...], vd_ref[...], e, ue], axis=1)
        h = _sp2(_mm(x, w1_ref[...]) + b1_ref[...])
        h = _sp2(_mm(h, w2_ref[...]) + b2_ref[...])
        em = _sp2(_mm(h, w3_ref[...]) + b3_ref[...])
        en = em + e
        em_ref[...] = em
        en_ref[...] = en
        if first_blk:
            _acc(q0_ref, _segsum(oh, e), pl.program_id(0) == 0)

    row = lambda w: pl.BlockSpec((T, w), lambda i: (i, 0))
    if first_blk:
        in_specs = [row(32), row(32), row(1), _full(ecnt), _full(u),
                    _full(enc[0]), _full(enc[1]), _full(enc[2]), _full(enc[3]),
                    _full(w1), _full(b1), _full(w2), _full(b2), _full(w3),
                    _full(b3)]
        args = (vs, vd, e_or_bd, ecnt, u, *enc, w1, b1, w2, b2, w3, b3)
        out_specs = [row(32), row(32), _cfix((G, 32))]
        out_shape = [jax.ShapeDtypeStruct((E, 32), F32),
                     jax.ShapeDtypeStruct((E, 32), F32),
                     jax.ShapeDtypeStruct((G, 32), F32)]
    else:
        in_specs = [row(32), row(32), row(32), _full(ecnt), _full(u),
                    _full(w1), _full(b1), _full(w2), _full(b2), _full(w3),
                    _full(b3)]
        args = (vs, vd, e_or_bd, ecnt, u, w1, b1, w2, b2, w3, b3)
        out_specs = [row(32), row(32)]
        out_shape = [jax.ShapeDtypeStruct((E, 32), F32),
                     jax.ShapeDtypeStruct((E, 32), F32)]
    return pl.pallas_call(
        body, grid=(grid,), in_specs=in_specs, out_specs=out_specs,
        out_shape=out_shape,
    )(*args)


def _node_block(v0, aggs, cnts, u, w1, b1, w2, b2, w3, b3,
                gsums, qe_prev, ce, uw1, ub1, uw2, ub2, uw3, ub3,
                gi3=None, vcnt_row=None, vcnt_col=None):
    N = v0.shape[0]
    G = u.shape[0]
    T = _TN
    grid = N // T
    first_blk = gi3 is not None

    def body(*refs):
        if first_blk:
            (v_ref, a0_ref, a1_ref, c0_ref, c1_ref, gi_ref, u_ref,
             w1_ref, b1_ref, w2_ref, b2_ref, w3_ref, b3_ref,
             sg0_ref, sg1_ref, qe_ref, ce_ref,
             uw1_ref, ub1_ref, uw2_ref, ub2_ref, uw3_ref, ub3_ref,
             vn_ref, sq_ref, un_ref, qe_out, cc_ref, cr_ref, sp_ref) = refs
            oh = _onehot(gi_ref[0], G)
        else:
            (v_ref, a0_ref, a1_ref, c0_ref, c1_ref, vc_ref, vcc_ref, u_ref,
             w1_ref, b1_ref, w2_ref, b2_ref, w3_ref, b3_ref,
             sg0_ref, sg1_ref, qe_ref, ce_ref,
             uw1_ref, ub1_ref, uw2_ref, ub2_ref, uw3_ref, ub3_ref,
             vn_ref, sq_ref, un_ref, qe_out, sp_ref) = refs
            oh = _onehot_bounds(vc_ref[...], T, G)
        cnt = jnp.maximum(c0_ref[0][:, 0:1] + c1_ref[0][:, 0:1], 1.0)
        em = (a0_ref[0] + a1_ref[0]) / cnt
        uv = _gmm(oh, u_ref[...])
        x = jnp.concatenate([v_ref[...], em, uv], axis=1)
        h = _sp2(_mm(x, w1_ref[...]) + b1_ref[...])
        h = _sp2(_mm(h, w2_ref[...]) + b2_ref[...])
        vm = _sp2(_mm(h, w3_ref[...]) + b3_ref[...])
        vn = vm + v_ref[...]
        vn_ref[...] = vn
        first = pl.program_id(0) == 0
        if first_blk:
            both = _segsum(oh, jnp.concatenate(
                [vm, vn, jnp.ones((T, 1), F32)], axis=1))
            sp_tot = _acc(sp_ref, both[:, 0:32], first)
            _acc(sq_ref, both[:, 32:64], first)
            cv_tot = _acc(cc_ref, both[:, 64:65], first)
            _acc(cr_ref, jnp.sum(oh, axis=0, keepdims=True), first)
        else:
            both = _segsum(oh, jnp.concatenate([vm, vn], axis=1))
            sp_tot = _acc(sp_ref, both[:, 0:32], first)
            _acc(sq_ref, both[:, 32:64], first)
            cv_tot = vcc_ref[...]

        @pl.when(pl.program_id(0) == grid - 1)
        def _():
            se = sg0_ref[0] + sg1_ref[0]
            qe_out[...] = qe_ref[...] + se
            me = se / jnp.maximum(ce_ref[...], 1.0)
            mv = sp_tot / jnp.maximum(cv_tot, 1.0)
            xs = jnp.concatenate([u_ref[...], me, mv], axis=1)
            hs = _sp2(_mm(xs, uw1_ref[...]) + ub1_ref[...])
            hs = _sp2(_mm(hs, uw2_ref[...]) + ub2_ref[...])
            un_ref[...] = _sp2(_mm(hs, uw3_ref[...]) + ub3_ref[...]) + u_ref[...]

    row = lambda w: pl.BlockSpec((T, w), lambda i: (i, 0))
    p0 = pl.BlockSpec((1, T, 32), lambda i: (0, i, 0))
    p1 = pl.BlockSpec((1, T, 32), lambda i: (1, i, 0))
    G32 = gsums.shape[2]
    g0 = pl.BlockSpec((1, G, G32), lambda i: (0, 0, 0))
    g1 = pl.BlockSpec((1, G, G32), lambda i: (1, 0, 0))
    w_specs = [_full(x) for x in (u, w1, b1, w2, b2, w3, b3)] + [g0, g1] + [
        _full(x) for x in (qe_prev, ce, uw1, ub1, uw2, ub2, uw3, ub3)]
    if first_blk:
        in_specs = ([row(32), p0, p1, p0, p1]
                    + [pl.BlockSpec((1, T, 1), lambda i: (i, 0, 0))] + w_specs)
        args = (v0, aggs, aggs, cnts, cnts, gi3, u, w1, b1, w2, b2, w3, b3,
                gsums, gsums, qe_prev, ce, uw1, ub1, uw2, ub2, uw3, ub3)
        out_specs = [row(32), _cfix((G, 32)), _cfix((G, 32)), _cfix((G, 32)),
                     _cfix((G, 1)), _cfix((1, G)), _cfix((G, 32))]
        out_shape = [jax.ShapeDtypeStruct((N, 32), F32),
                     jax.ShapeDtypeStruct((G, 32), F32),
                     jax.ShapeDtypeStruct((G, 32), F32),
                     jax.ShapeDtypeStruct((G, 32), F32),
                     jax.ShapeDtypeStruct((G, 1), F32),
                     jax.ShapeDtypeStruct((1, G), F32),
                     jax.ShapeDtypeStruct((G, 32), F32)]
    else:
        in_specs = ([row(32), p0, p1, p0, p1]
                    + [_full(vcnt_row), _full(vcnt_col)] + w_specs)
        args = (v0, aggs, aggs, cnts, cnts, vcnt_row, vcnt_col, u,
                w1, b1, w2, b2, w3, b3, gsums, gsums, qe_prev, ce,
                uw1, ub1, uw2, ub2, uw3, ub3)
        out_specs = [row(32), _cfix((G, 32)), _cfix((G, 32)), _cfix((G, 32)),
                     _cfix((G, 32))]
        out_shape = [jax.ShapeDtypeStruct((N, 32), F32),
                     jax.ShapeDtypeStruct((G, 32), F32),
                     jax.ShapeDtypeStruct((G, 32), F32),
                     jax.ShapeDtypeStruct((G, 32), F32),
                     jax.ShapeDtypeStruct((G, 32), F32)]
    return pl.pallas_call(
        body, grid=(grid,), in_specs=in_specs, out_specs=out_specs,
        out_shape=out_shape,
    )(*args)


def _s2s_pass_a(x, h_or_lstm, cnt_row):
    R = x.shape[0]
    G = cnt_row.shape[1]
    T = _TE if R % _TE == 0 else _TN
    grid = R // T
    with_lstm = isinstance(h_or_lstm, tuple)

    def lstm_half(r1, w_ref, b_ref, d):
        g = _mm(r1, w_ref[d:, :]) + b_ref[...]
        i = jax.nn.sigmoid(g[:, 0:d])
        gg = jnp.tanh(g[:, 2 * d:3 * d])
        o = jax.nn.sigmoid(g[:, 3 * d:4 * d])
        return o * jnp.tanh(i * gg)

    def body(*refs):
        if with_lstm:
            (x_ref, ec_ref, sv_ref, cv_ref, se_ref, ce_ref,
             wn_ref, bn_ref, we_ref, be_ref,
             m_ref, hn_ref, he_ref, h_scr) = refs

            @pl.when(pl.program_id(0) == 0)
            def _():
                rn = sv_ref[...] / jnp.maximum(cv_ref[...], 1.0)
                re = se_ref[...] / jnp.maximum(ce_ref[...], 1.0)
                hn = lstm_half(rn, wn_ref, bn_ref, 32)
                hn_ref[...] = hn
                he_ref[...] = lstm_half(re, we_ref, be_ref, 32)
                h_scr[...] = hn

            h = h_scr[...]
        else:
            x_ref, ec_ref, h_ref, m_ref = refs
            h = h_ref[...]
        oh = _onehot_bounds(ec_ref[...], T, G)
        hseg = _gmm(oh, h)
        s = jnp.sum(x_ref[...] * hseg, axis=1, keepdims=True)
        mp = jnp.max(jnp.where(oh > 0, s, -1e30), axis=0, keepdims=True)
        first = pl.program_id(0) == 0
        m_ref[...] = jnp.where(first, mp, jnp.maximum(m_ref[...], mp))

    row32 = pl.BlockSpec((T, 32), lambda i: (i, 0))
    if with_lstm:
        sv, cv, se, ce, wn, bn, we, be = h_or_lstm
        return pl.pallas_call(
            body, grid=(grid,),
            in_specs=[row32, _full(cnt_row), _full(sv), _full(cv), _full(se),
                      _full(ce), _full(wn), _full(bn), _full(we), _full(be)],
            out_specs=[_cfix((1, G)), _cfix((G, 32)), _cfix((G, 32))],
            out_shape=[jax.ShapeDtypeStruct((1, G), F32),
                       jax.ShapeDtypeStruct((G, 32), F32),
                       jax.ShapeDtypeStruct((G, 32), F32)],
            scratch_shapes=[pltpu.VMEM((G, 32), F32)],
        )(x, cnt_row, sv, cv, se, ce, wn, bn, we, be)
    h = h_or_lstm
    return pl.pallas_call(
        body, grid=(grid,),
        in_specs=[row32, _full(cnt_row), _full(h)],
        out_specs=_cfix((1, G)),
        out_shape=jax.ShapeDtypeStruct((1, G), F32),
    )(x, cnt_row, h)


def _s2s_pass_b(x, h, cnt_row, m, readout=None):
    R = x.shape[0]
    G = m.shape[1]
    T = _TE if R % _TE == 0 else _TN
    grid = R // T

    def body(*refs):
        if readout is None:
            x_ref, ec_ref, h_ref, m_ref, d_ref, n_ref = refs
        else:
            (x_ref, ec_ref, h_ref, m_ref,
             hn_ref, nn_ref, dnr, u_ref,
             ow1r, ob1r, ow2r, ob2r, opwr, opbr,
             latr, lw1r, lb1r, lw2r, lb2r, lw3r, lb3r,
             fprer, pw1r, pb1r, pw2r, pb2r, pw3r, pb3r,
             sgr, embr, woutr, boutr, out_ref, d_ref, n_ref) = refs
        oh = _onehot_bounds(ec_ref[...], T, G)
        hseg = _gmm(oh, h_ref[...])
        xv = x_ref[...]
        s = jnp.sum(xv * hseg, axis=1, keepdims=True)
        mseg = jnp.sum(oh * m_ref[...], axis=1, keepdims=True)
        a = jnp.exp(s - mseg)
        both = _segsum(oh, jnp.concatenate([a, a * xv], axis=1))
        first = pl.program_id(0) == 0
        de = _acc(d_ref, both[:, 0:1], first)
        ne = _acc(n_ref, both[:, 1:33], first)
        if readout is not None:
            @pl.when(pl.program_id(0) == grid - 1)
            def _():
                rn = nn_ref[...] / (dnr[...] + 1e-12)
                re = ne / (de + 1e-12)
                z = jnp.concatenate(
                    [hn_ref[...], rn, h_ref[...], re, u_ref[...]], axis=1)
                z = _sp2(_mm(z, ow1r[...]) + ob1r[...])
                z = _sp2(_mm(z, ow2r[...]) + ob2r[...])
                xa = _mm(z, opwr[...]) + opbr[...]
                xl = _elu(_mm(latr[...], lw1r[...]) + lb1r[...])
                xl = _elu(_mm(xl, lw2r[...]) + lb2r[...])
                xl = _elu(_mm(xl, lw3r[...]) + lb3r[...])
                xp = _elu(_mm(fprer[...], pw1r[...]) + pb1r[...])
                xp = _elu(_mm(xp, pw2r[...]) + pb2r[...])
                xp = _elu(_mm(xp, pw3r[...]) + pb3r[...])
                ohg = _onehot(sgr[0], 256)
                xs = _gmm(ohg, embr[...])
                fx = jnp.concatenate([xa, xl, xp, xs], axis=1)
                out_ref[...] = _mm(fx, woutr[...]) + boutr[...]

    if readout is None:
        return pl.pallas_call(
            body, grid=(grid,),
            in_specs=[pl.BlockSpec((T, 32), lambda i: (i, 0)),
                      _full(cnt_row), _full(h), _full(m)],
            out_specs=[_cfix((G, 1)), _cfix((G, 32))],
            out_shape=[jax.ShapeDtypeStruct((G, 1), F32),
                       jax.ShapeDtypeStruct((G, 32), F32)],
        )(x, cnt_row, h, m)
    ro = readout
    sg3 = ro[24]
    return pl.pallas_call(
        body, grid=(grid,),
        in_specs=([pl.BlockSpec((T, 32), lambda i: (i, 0)),
                   _full(cnt_row), _full(h), _full(m)]
                  + [_full(a) for a in ro[:24]]
                  + [pl.BlockSpec(sg3.shape, lambda i: (0, 0, 0))]
                  + [_full(a) for a in ro[25:]]),
        out_specs=[_cfix((G, 128)), _cfix((G, 1)), _cfix((G, 32))],
        out_shape=[jax.ShapeDtypeStruct((G, 128), F32),
                   jax.ShapeDtypeStruct((G, 1), F32),
                   jax.ShapeDtypeStruct((G, 32), F32)],
    )(x, cnt_row, h, m, *ro)[0]


def _sc_gather_pair(table, isrc, idst):
    E = isrc.shape[0]
    D = table.shape[1]
    epw = E // _NW
    ch = 400
    nch = epw // ch
    mesh = plsc.VectorSubcoreMesh(core_axis_name="c", subcore_axis_name="s")

    @functools.partial(
        pl.kernel,
        out_type=[
            jax.ShapeDtypeStruct((E, D), F32),
            jax.ShapeDtypeStruct((E, D), F32),
        ],
        mesh=mesh,
        compiler_params=pltpu.CompilerParams(use_tc_tiling_on_sc=False),
        scratch_types=[
            [pltpu.VMEM((ch,), jnp.int32)] * 2,
            [pltpu.VMEM((ch, D), F32)] * 2,
            [pltpu.VMEM((ch,), jnp.int32)] * 2,
            [pltpu.VMEM((ch, D), F32)] * 2,
            [pltpu.SemaphoreType.DMA] * 2,
            [pltpu.SemaphoreType.DMA] * 2,
        ],
    )
    def k(table_h, isrc_h, idst_h, osrc_h, odst_h, iv1, rv1, iv2, rv2, s1, s2):
        wid = lax.axis_index("s") * _NC + lax.axis_index("c")
        base = wid * epw

        def load_and_fire(c):
            b = c % 2
            off = base + c * ch
            pltpu.sync_copy(isrc_h.at[pl.ds(off, ch)], iv1[b])
            pltpu.sync_copy(idst_h.at[pl.ds(off, ch)], iv2[b])
            g1 = pltpu.async_copy(table_h.at[iv1[b]], rv1[b], s1[b])
            g2 = pltpu.async_copy(table_h.at[iv2[b]], rv2[b], s2[b])
            return g1, g2

        def drain(c, g1, g2):
            b = c % 2
            off = base + c * ch
            g1.wait()
            g2.wait()
            pltpu.sync_copy(rv1[b], osrc_h.at[pl.ds(off, ch)])
            pltpu.sync_copy(rv2[b], odst_h.at[pl.ds(off, ch)])

        pend = load_and_fire(0)
        for c in range(1, nch):
            nxt = load_and_fire(c)
            drain(c - 1, *pend)
            pend = nxt
        drain(nch - 1, *pend)

    return k(table, isrc, idst)


def _sc_scatter_add(data, idx, gidx, nrows, G, zeros):
    E = idx.shape[0]
    D = data.shape[1]
    epw = E // _NW
    ch = 2000
    nch = epw // ch
    stripe = nrows // _NS
    gstripe = G // _NS
    mesh = plsc.VectorSubcoreMesh(core_axis_name="c", subcore_axis_name="s")

    @functools.partial(
        pl.kernel,
        out_type=[jax.ShapeDtypeStruct((_NC, nrows, D), F32),
                  jax.ShapeDtypeStruct((_NC, G, D), F32)],
        mesh=mesh,
        compiler_params=pltpu.CompilerParams(use_tc_tiling_on_sc=False),
        scratch_types=[
            pltpu.VMEM((ch,), jnp.int32),
            pltpu.VMEM((ch,), jnp.int32),
            pltpu.VMEM((ch, D), F32),
            pltpu.VMEM_SHARED((nrows, D), F32),
            pltpu.VMEM_SHARED((G, D), F32),
        ],
    )
    def k(data_h, idx_h, gidx_h, zeros_h, out_h, gout_h, iv, gv, dv, acc, gacc):
        cid = lax.axis_index("c")
        sid = lax.axis_index("s")
        wid = sid * _NC + cid
        pltpu.sync_copy(zeros_h.at[pl.ds(sid * stripe, stripe)],
                        acc.at[pl.ds(sid * stripe, stripe)])
        pltpu.sync_copy(zeros_h.at[pl.ds(sid * gstripe, gstripe)],
                        gacc.at[pl.ds(sid * gstripe, gstripe)])
        plsc.subcore_barrier()
        base = wid * epw
        for c in range(nch):
            off = base + c * ch
            pltpu.sync_copy(idx_h.at[pl.ds(off, ch)], iv)
            pltpu.sync_copy(gidx_h.at[pl.ds(off, ch)], gv)
            pltpu.sync_copy(data_h.at[pl.ds(off, ch)], dv)
            pltpu.sync_copy(dv, acc.at[iv], add=True)
            pltpu.sync_copy(dv, gacc.at[gv], add=True)
        plsc.subcore_barrier()
        pltpu.sync_copy(acc.at[pl.ds(sid * stripe, stripe)],
                        out_h.at[cid, pl.ds(sid * stripe, stripe)])
        pltpu.sync_copy(gacc.at[pl.ds(sid * gstripe, gstripe)],
                        gout_h.at[cid, pl.ds(sid * gstripe, gstripe)])

    return k(data, idx, gidx, zeros)


def _sc_scatter_ones(idx, gidx, nrows, G, zeros, ones_ch):
    E = idx.shape[0]
    ch, D = ones_ch.shape
    epw = E // _NW
    nch = epw // ch
    stripe = nrows // _NS
    gstripe = G // _NS
    mesh = plsc.VectorSubcoreMesh(core_axis_name="c", subcore_axis_name="s")

    @functools.partial(
        pl.kernel,
        out_type=[jax.ShapeDtypeStruct((_NC, nrows, D), F32),
                  jax.ShapeDtypeStruct((_NC, G, D), F32)],
        mesh=mesh,
        compiler_params=pltpu.CompilerParams(use_tc_tiling_on_sc=False),
        scratch_types=[
            pltpu.VMEM((ch,), jnp.int32),
            pltpu.VMEM((ch,), jnp.int32),
            pltpu.VMEM((ch, D), F32),
            pltpu.VMEM_SHARED((nrows, D), F32),
            pltpu.VMEM_SHARED((G, D), F32),
        ],
    )
    def k(idx_h, gidx_h, zeros_h, ones_h, out_h, gout_h, iv, gv, dv, acc, gacc):
        cid = lax.axis_index("c")
        sid = lax.axis_index("s")
        wid = sid * _NC + cid
        pltpu.sync_copy(zeros_h.at[pl.ds(sid * stripe, stripe)],
                        acc.at[pl.ds(sid * stripe, stripe)])
        pltpu.sync_copy(zeros_h.at[pl.ds(sid * gstripe, gstripe)],
                        gacc.at[pl.ds(sid * gstripe, gstripe)])
        pltpu.sync_copy(ones_h, dv)
        plsc.subcore_barrier()
        base = wid * epw
        for c in range(nch):
            off = base + c * ch
            pltpu.sync_copy(idx_h.at[pl.ds(off, ch)], iv)
            pltpu.sync_copy(gidx_h.at[pl.ds(off, ch)], gv)
            pltpu.sync_copy(dv, acc.at[iv], add=True)
            pltpu.sync_copy(dv, gacc.at[gv], add=True)
        plsc.subcore_barrier()
        pltpu.sync_copy(acc.at[pl.ds(sid * stripe, stripe)],
                        out_h.at[cid, pl.ds(sid * stripe, stripe)])
        pltpu.sync_copy(gacc.at[pl.ds(sid * gstripe, gstripe)],
                        gout_h.at[cid, pl.ds(sid * gstripe, gstripe)])

    return k(idx, gidx, zeros, ones_ch)


def kernel(edge_index, bond_dist, node_feat, state_attr, node_graph_idx,
           edge_graph_idx, sg, lattice, fpretrain, params):
    E = bond_dist.shape[0]
    N = node_feat.shape[0]
    G = state_attr.shape[0]

    src = edge_index[0].astype(jnp.int32)
    dst = edge_index[1].astype(jnp.int32)
    gi_n3 = node_graph_idx.astype(jnp.int32).reshape(N // _TN, _TN, 1)
    nf3 = node_feat.astype(jnp.int32).reshape(N // _TN, _TN, 1)
    sg3 = sg.astype(jnp.int32).reshape(1, G, 1)
    bd = bond_dist.reshape(E, 1)

    p = params
    rb = lambda b: b.reshape(1, -1)

    (ew1, eb1), (ew2, eb2) = p['edge_enc']
    enc = (jnp.pad(ew1, ((0, 28), (0, 0))), rb(eb1), ew2, rb(eb2))

    (nw1, nb1), (nw2, nb2) = p['node_enc']
    (sw1, sb1), (sw2, sb2) = p['state_enc']
    embp = jnp.pad(p['emb_atom'], ((0, 33), (0, 0)))
    v, u = _node_encoder(nf3, embp, nw1, rb(nb1), nw2, rb(nb2),
                         state_attr, sw1, rb(sb1), sw2, rb(sb2))

    zeros_n = jnp.zeros((N, 32), F32)
    gie = edge_graph_idx.astype(jnp.int32)
    counts, gcounts = _sc_scatter_ones(dst, gie, N, G, zeros_n,
                                       jnp.ones((2000, 32), F32))
    ecnt_row = (gcounts[0, :, 0] + gcounts[1, :, 0]).reshape(1, G)
    ecnt_col = ecnt_row.reshape(G, 1)

    e_in = bd
    vcnt_row = vcnt_col = None
    qe = None
    seg_v_post = None
    for bi, blk in enumerate(p['blocks']):
        (bw1, bb1), (bw2, bb2), (bw3, bb3) = blk['edge']
        vs, vd = _sc_gather_pair(v, src, dst)
        if bi == 0:
            em, en, q0 = _edge_block(
                vs, vd, e_in, u, bw1, rb(bb1), bw2, rb(bb2), bw3, rb(bb3),
                enc=enc, ecnt=ecnt_row)
            qe = q0
        else:
            em, en = _edge_block(
                vs, vd, e_in, u, bw1, rb(bb1), bw2, rb(bb2), bw3, rb(bb3),
                ecnt=ecnt_row)
        aggs, gsums = _sc_scatter_add(em, dst, gie, N, G, zeros_n)
        (vw1, vb1), (vw2, vb2), (vw3, vb3) = blk['node']
        (uw1, ub1), (uw2, ub2), (uw3, ub3) = blk['state']
        if bi == 0:
            vn, sv_post, un, qe, vcnt_col, vcnt_row, _sv_pre = _node_block(
                v, aggs, counts, u,
                vw1, rb(vb1), vw2, rb(vb2), vw3, rb(vb3),
                gsums, qe, ecnt_col,
                uw1, rb(ub1), uw2, rb(ub2), uw3, rb(ub3),
                gi3=gi_n3)
        else:
            vn, sv_post, un, qe, _sv_pre = _node_block(
                v, aggs, counts, u,
                vw1, rb(vb1), vw2, rb(vb2), vw3, rb(vb3),
                gsums, qe, ecnt_col,
                uw1, rb(ub1), uw2, rb(ub2), uw3, rb(ub3),
                vcnt_row=vcnt_row, vcnt_col=vcnt_col)
        e_in, v, u = en, vn, un
        seg_v_post = sv_post

    mn, hn, he = _s2s_pass_a(
        v, (seg_v_post, vcnt_col, qe, ecnt_col,
            p['s2s_node']['Wih'], rb(p['s2s_node']['b']),
            p['s2s_edge']['Wih'], rb(p['s2s_edge']['b'])), vcnt_row)
    me_ = _s2s_pass_a(e_in, he, ecnt_row)
    den_n, num_n = _s2s_pass_b(v, hn, vcnt_row, mn)

    (ow1, ob1), (ow2, ob2) = p['out_mlp']
    opw, opb = p['out_proj']
    (lw1, lb1), (lw2, lb2), (lw3, lb3) = p['emb_lattice']
    (pw1, pb1), (pw2, pb2), (pw3, pb3) = p['emb_pretrain']
    embsg = jnp.pad(p['emb_sg'], ((0, 26), (0, 0)))
    wout, bout = p['output_layer']

    return _s2s_pass_b(
        e_in, he, ecnt_row, me_,
        readout=(hn, num_n, den_n, u,
                 ow1, rb(ob1), ow2, rb(ob2), opw, rb(opb),
                 lattice, lw1, rb(lb1), lw2, rb(lb2), lw3, rb(lb3),
                 fpretrain, pw1, rb(pb1), pw2, rb(pb2), pw3, rb(pb3),
                 sg3, embsg, wout, rb(bout)))


# speedup vs baseline: 1.0699x; 1.0284x over previous
import functools

import jax
import jax.numpy as jnp
from jax import lax
from jax.experimental import pallas as pl
from jax.experimental.pallas import tpu as pltpu
from jax.experimental.pallas import tpu_sc as plsc

F32 = jnp.float32
BF16 = jnp.bfloat16
_LOG2 = 0.6931471805599453

_NC = 2
_NS = 16
_NW = _NC * _NS

_TE = 6400
_TN = 2000


def _split(x):
    hi = x.astype(BF16)
    lo = (x - hi.astype(F32)).astype(BF16)
    return hi, lo


def _dot3(a, b, dims):
    ah, al = _split(a)
    bh, bl = _split(b)
    d = lambda p, q: lax.dot_general(p, q, (dims, ((), ())),
                                     preferred_element_type=F32)
    return d(ah, bh) + d(ah, bl) + d(al, bh)


def _mm(a, b):
    return _dot3(a, b, ((1,), (0,)))


def _gmm(oh, t):
    ohh = oh.astype(BF16)
    th, tl = _split(t)
    d = lambda p, q: lax.dot_general(p, q, (((1,), (0,)), ((), ())),
                                     preferred_element_type=F32)
    return d(ohh, th) + d(ohh, tl)


def _segsum(oh, x):
    ohh = oh.astype(BF16)
    xh, xl = _split(x)
    d = lambda p, q: lax.dot_general(p, q, (((0,), (0,)), ((), ())),
                                     preferred_element_type=F32)
    return d(ohh, xh) + d(ohh, xl)


def _sp2(x):
    return jnp.maximum(x, 0.0) + jnp.log1p(jnp.exp(-jnp.abs(x))) - _LOG2


def _elu(x):
    return jnp.where(x > 0, x, jnp.exp(jnp.minimum(x, 0.0)) - 1.0)


def _onehot(ids_col, n):
    return (ids_col == lax.broadcasted_iota(jnp.int32, (1, n), 1)).astype(F32)


def _onehot_bounds(cnt_row, T, G):
    tri = (lax.broadcasted_iota(jnp.int32, (G, G), 0)
           <= lax.broadcasted_iota(jnp.int32, (G, G), 1)).astype(F32)
    ic = _gmm(cnt_row, tri)
    lo = ic - cnt_row
    r = (pl.program_id(0) * T
         + lax.broadcasted_iota(jnp.int32, (T, 1), 0)).astype(F32)
    return ((r >= lo) & (r < ic)).astype(F32)


def _acc(ref, val, first):
    tot = jnp.where(first, val, ref[...] + val)
    ref[...] = tot
    return tot


def _full(a):
    return pl.BlockSpec(a.shape, lambda i: tuple(0 for _ in a.shape))


def _cfix(shape):
    return pl.BlockSpec(shape, lambda i: tuple(0 for _ in shape))


def _node_encoder(nf3, embp, w1, b1, w2, b2, sa, sw1, sb1, sw2, sb2):
    grid, T, _ = nf3.shape
    N = grid * T
    G = sa.shape[0]

    def body(nf_ref, emb_ref, w1_ref, b1_ref, w2_ref, b2_ref,
             sa_ref, sw1_ref, sb1_ref, sw2_ref, sb2_ref, out_ref, u_ref):
        oh = _onehot(nf_ref[0], 128)
        vemb = _gmm(oh, emb_ref[...])
        h = _sp2(_mm(vemb, w1_ref[...]) + b1_ref[...])
        out_ref[...] = _sp2(_mm(h, w2_ref[...]) + b2_ref[...])

        @pl.when(pl.program_id(0) == 0)
        def _():
            hs = _sp2(_mm(sa_ref[...], sw1_ref[...]) + sb1_ref[...])
            u_ref[...] = _sp2(_mm(hs, sw2_ref[...]) + sb2_ref[...])

    return pl.pallas_call(
        body,
        grid=(grid,),
        in_specs=[
            pl.BlockSpec((1, T, 1), lambda i: (i, 0, 0)),
            _full(embp), _full(w1), _full(b1), _full(w2), _full(b2),
            _full(sa), _full(sw1), _full(sb1), _full(sw2), _full(sb2),
        ],
        out_specs=[pl.BlockSpec((T, 32), lambda i: (i, 0)),
                   _cfix((G, 32))],
        out_shape=[jax.ShapeDtypeStruct((N, 32), F32),
                   jax.ShapeDtypeStruct((G, 32), F32)],
    )(nf3, embp, w1, b1, w2, b2, sa, sw1, sb1, sw2, sb2)


def _edge_block(vs, vd, e_or_bd, u, w1, b1, w2, b2, w3, b3,
                enc=None, ecnt=None):
    E = vs.shape[0]
    G = u.shape[0]
    T = _TE
    grid = E // T
    first_blk = enc is not None

    def body(*refs):
        if first_blk:
            (vs_ref, vd_ref, e_ref, ec_ref, u_ref,
             cw1_ref, cb1_ref, cw2_ref, cb2_ref,
             w1_ref, b1_ref, w2_ref, b2_ref, w3_ref, b3_ref,
             em_ref, en_ref, q0_ref) = refs
            d = e_ref[...]
            cc = lax.broadcasted_iota(jnp.int32, (1, 128), 1).astype(F32) * (5.0 / 99.0)
            rbf = jnp.exp(-((d - cc) ** 2) * 4.0)
            eh = _sp2(_mm(rbf, cw1_ref[...]) + cb1_ref[...])
            e = _sp2(_mm(eh, cw2_ref[...]) + cb2_ref[...])
        else:
            (vs_ref, vd_ref, e_ref, ec_ref, u_ref,
             w1_ref, b1_ref, w2_ref, b2_ref, w3_ref, b3_ref,
             em_ref, en_ref) = refs
            e = e_ref[...]
        oh = _onehot_bounds(ec_ref[...], T, G)
        ue = _gmm(oh, u_ref[...])
        x = jnp.concatenate([vs_ref[...], vd_ref[...], e, ue], axis=1)
        h = _sp2(_mm(x, w1_ref[...]) + b1_ref[...])
        h = _sp2(_mm(h, w2_ref[...]) + b2_ref[...])
        em = _sp2(_mm(h, w3_ref[...]) + b3_ref[...])
        en = em + e
        em_ref[...] = em
        en_ref[...] = en
        if first_blk:
            _acc(q0_ref, _segsum(oh, e), pl.program_id(0) == 0)

    row = lambda w: pl.BlockSpec((T, w), lambda i: (i, 0))
    if first_blk:
        in_specs = [row(32), row(32), row(1), _full(ecnt), _full(u),
                    _full(enc[0]), _full(enc[1]), _full(enc[2]), _full(enc[3]),
                    _full(w1), _full(b1), _full(w2), _full(b2), _full(w3),
                    _full(b3)]
        args = (vs, vd, e_or_bd, ecnt, u, *enc, w1, b1, w2, b2, w3, b3)
        out_specs = [row(32), row(32), _cfix((G, 32))]
        out_shape = [jax.ShapeDtypeStruct((E, 32), F32),
                     jax.ShapeDtypeStruct((E, 32), F32),
                     jax.ShapeDtypeStruct((G, 32), F32)]
    else:
        in_specs = [row(32), row(32), row(32), _full(ecnt), _full(u),
                    _full(w1), _full(b1), _full(w2), _full(b2), _full(w3),
                    _full(b3)]
        args = (vs, vd, e_or_bd, ecnt, u, w1, b1, w2, b2, w3, b3)
        out_specs = [row(32), row(32)]
        out_shape = [jax.ShapeDtypeStruct((E, 32), F32),
                     jax.ShapeDtypeStruct((E, 32), F32)]
    return pl.pallas_call(
        body, grid=(grid,), in_specs=in_specs, out_specs=out_specs,
        out_shape=out_shape,
    )(*args)


def _node_block(v0, aggs, cnts, u, w1, b1, w2, b2, w3, b3,
                gsums, qe_prev, ce, uw1, ub1, uw2, ub2, uw3, ub3,
                gi3=None, vcnt_row=None, vcnt_col=None):
    N = v0.shape[0]
    G = u.shape[0]
    T = _TN
    grid = N // T
    first_blk = gi3 is not None

    def body(*refs):
        if first_blk:
            (v_ref, a0_ref, a1_ref, c0_ref, c1_ref, gi_ref, u_ref,
             w1_ref, b1_ref, w2_ref, b2_ref, w3_ref, b3_ref,
             sg0_ref, sg1_ref, qe_ref, ce_ref,
             uw1_ref, ub1_ref, uw2_ref, ub2_ref, uw3_ref, ub3_ref,
             vn_ref, sq_ref, un_ref, qe_out, cc_ref, cr_ref, sp_ref) = refs
            oh = _onehot(gi_ref[0], G)
        else:
            (v_ref, a0_ref, a1_ref, c0_ref, c1_ref, vc_ref, vcc_ref, u_ref,
             w1_ref, b1_ref, w2_ref, b2_ref, w3_ref, b3_ref,
             sg0_ref, sg1_ref, qe_ref, ce_ref,
             uw1_ref, ub1_ref, uw2_ref, ub2_ref, uw3_ref, ub3_ref,
             vn_ref, sq_ref, un_ref, qe_out, sp_ref) = refs
            oh = _onehot_bounds(vc_ref[...], T, G)
        cnt = jnp.maximum(c0_ref[0][:, 0:1] + c1_ref[0][:, 0:1], 1.0)
        em = (a0_ref[0] + a1_ref[0]) / cnt
        uv = _gmm(oh, u_ref[...])
        x = jnp.concatenate([v_ref[...], em, uv], axis=1)
        h = _sp2(_mm(x, w1_ref[...]) + b1_ref[...])
        h = _sp2(_mm(h, w2_ref[...]) + b2_ref[...])
        vm = _sp2(_mm(h, w3_ref[...]) + b3_ref[...])
        vn = vm + v_ref[...]
        vn_ref[...] = vn
        first = pl.program_id(0) == 0
        if first_blk:
            both = _segsum(oh, jnp.concatenate(
                [vm, vn, jnp.ones((T, 1), F32)], axis=1))
            sp_tot = _acc(sp_ref, both[:, 0:32], first)
            _acc(sq_ref, both[:, 32:64], first)
            cv_tot = _acc(cc_ref, both[:, 64:65], first)
            _acc(cr_ref, jnp.sum(oh, axis=0, keepdims=True), first)
        else:
            both = _segsum(oh, jnp.concatenate([vm, vn], axis=1))
            sp_tot = _acc(sp_ref, both[:, 0:32], first)
            _acc(sq_ref, both[:, 32:64], first)
            cv_tot = vcc_ref[...]

        @pl.when(pl.program_id(0) == grid - 1)
        def _():
            se = sg0_ref[0] + sg1_ref[0]
            qe_out[...] = qe_ref[...] + se
            me = se / jnp.maximum(ce_ref[...], 1.0)
            mv = sp_tot / jnp.maximum(cv_tot, 1.0)
            xs = jnp.concatenate([u_ref[...], me, mv], axis=1)
            hs = _sp2(_mm(xs, uw1_ref[...]) + ub1_ref[...])
            hs = _sp2(_mm(hs, uw2_ref[...]) + ub2_ref[...])
            un_ref[...] = _sp2(_mm(hs, uw3_ref[...]) + ub3_ref[...]) + u_ref[...]

    row = lambda w: pl.BlockSpec((T, w), lambda i: (i, 0))
    p0 = pl.BlockSpec((1, T, 32), lambda i: (0, i, 0))
    p1 = pl.BlockSpec((1, T, 32), lambda i: (1, i, 0))
    G32 = gsums.shape[2]
    g0 = pl.BlockSpec((1, G, G32), lambda i: (0, 0, 0))
    g1 = pl.BlockSpec((1, G, G32), lambda i: (1, 0, 0))
    w_specs = [_full(x) for x in (u, w1, b1, w2, b2, w3, b3)] + [g0, g1] + [
        _full(x) for x in (qe_prev, ce, uw1, ub1, uw2, ub2, uw3, ub3)]
    if first_blk:
        in_specs = ([row(32), p0, p1, p0, p1]
                    + [pl.BlockSpec((1, T, 1), lambda i: (i, 0, 0))] + w_specs)
        args = (v0, aggs, aggs, cnts, cnts, gi3, u, w1, b1, w2, b2, w3, b3,
                gsums, gsums, qe_prev, ce, uw1, ub1, uw2, ub2, uw3, ub3)
        out_specs = [row(32), _cfix((G, 32)), _cfix((G, 32)), _cfix((G, 32)),
                     _cfix((G, 1)), _cfix((1, G)), _cfix((G, 32))]
        out_shape = [jax.ShapeDtypeStruct((N, 32), F32),
                     jax.ShapeDtypeStruct((G, 32), F32),
                     jax.ShapeDtypeStruct((G, 32), F32),
                     jax.ShapeDtypeStruct((G, 32), F32),
                     jax.ShapeDtypeStruct((G, 1), F32),
                     jax.ShapeDtypeStruct((1, G), F32),
                     jax.ShapeDtypeStruct((G, 32), F32)]
    else:
        in_specs = ([row(32), p0, p1, p0, p1]
                    + [_full(vcnt_row), _full(vcnt_col)] + w_specs)
        args = (v0, aggs, aggs, cnts, cnts, vcnt_row, vcnt_col, u,
                w1, b1, w2, b2, w3, b3, gsums, gsums, qe_prev, ce,
                uw1, ub1, uw2, ub2, uw3, ub3)
        out_specs = [row(32), _cfix((G, 32)), _cfix((G, 32)), _cfix((G, 32)),
                     _cfix((G, 32))]
        out_shape = [jax.ShapeDtypeStruct((N, 32), F32),
                     jax.ShapeDtypeStruct((G, 32), F32),
                     jax.ShapeDtypeStruct((G, 32), F32),
                     jax.ShapeDtypeStruct((G, 32), F32),
                     jax.ShapeDtypeStruct((G, 32), F32)]
    return pl.pallas_call(
        body, grid=(grid,), in_specs=in_specs, out_specs=out_specs,
        out_shape=out_shape,
    )(*args)


def _s2s_pass_a(x, h_or_lstm, cnt_row):
    R = x.shape[0]
    G = cnt_row.shape[1]
    T = _TE if R % _TE == 0 else _TN
    grid = R // T
    with_lstm = isinstance(h_or_lstm, tuple)

    def lstm_half(r1, w_ref, b_ref, d):
        g = _mm(r1, w_ref[d:, :]) + b_ref[...]
        i = jax.nn.sigmoid(g[:, 0:d])
        gg = jnp.tanh(g[:, 2 * d:3 * d])
        o = jax.nn.sigmoid(g[:, 3 * d:4 * d])
        return o * jnp.tanh(i * gg)

    def body(*refs):
        if with_lstm:
            (x_ref, ec_ref, sv_ref, cv_ref, se_ref, ce_ref,
             wn_ref, bn_ref, we_ref, be_ref,
             m_ref, hn_ref, he_ref, h_scr) = refs

            @pl.when(pl.program_id(0) == 0)
            def _():
                rn = sv_ref[...] / jnp.maximum(cv_ref[...], 1.0)
                re = se_ref[...] / jnp.maximum(ce_ref[...], 1.0)
                hn = lstm_half(rn, wn_ref, bn_ref, 32)
                hn_ref[...] = hn
                he_ref[...] = lstm_half(re, we_ref, be_ref, 32)
                h_scr[...] = hn

            h = h_scr[...]
        else:
            x_ref, ec_ref, h_ref, m_ref = refs
            h = h_ref[...]
        oh = _onehot_bounds(ec_ref[...], T, G)
        hseg = _gmm(oh, h)
        s = jnp.sum(x_ref[...] * hseg, axis=1, keepdims=True)
        mp = jnp.max(jnp.where(oh > 0, s, -1e30), axis=0, keepdims=True)
        first = pl.program_id(0) == 0
        m_ref[...] = jnp.where(first, mp, jnp.maximum(m_ref[...], mp))

    row32 = pl.BlockSpec((T, 32), lambda i: (i, 0))
    if with_lstm:
        sv, cv, se, ce, wn, bn, we, be = h_or_lstm
        return pl.pallas_call(
            body, grid=(grid,),
            in_specs=[row32, _full(cnt_row), _full(sv), _full(cv), _full(se),
                      _full(ce), _full(wn), _full(bn), _full(we), _full(be)],
            out_specs=[_cfix((1, G)), _cfix((G, 32)), _cfix((G, 32))],
            out_shape=[jax.ShapeDtypeStruct((1, G), F32),
                       jax.ShapeDtypeStruct((G, 32), F32),
                       jax.ShapeDtypeStruct((G, 32), F32)],
            scratch_shapes=[pltpu.VMEM((G, 32), F32)],
        )(x, cnt_row, sv, cv, se, ce, wn, bn, we, be)
    h = h_or_lstm
    return pl.pallas_call(
        body, grid=(grid,),
        in_specs=[row32, _full(cnt_row), _full(h)],
        out_specs=_cfix((1, G)),
        out_shape=jax.ShapeDtypeStruct((1, G), F32),
    )(x, cnt_row, h)


def _s2s_pass_b(x, h, cnt_row, m, readout=None):
    R = x.shape[0]
    G = m.shape[1]
    T = _TE if R % _TE == 0 else _TN
    grid = R // T

    def body(*refs):
        if readout is None:
            x_ref, ec_ref, h_ref, m_ref, d_ref, n_ref = refs
        else:
            (x_ref, ec_ref, h_ref, m_ref,
             hn_ref, nn_ref, dnr, u_ref,
             ow1r, ob1r, ow2r, ob2r, opwr, opbr,
             latr, lw1r, lb1r, lw2r, lb2r, lw3r, lb3r,
             fprer, pw1r, pb1r, pw2r, pb2r, pw3r, pb3r,
             sgr, embr, woutr, boutr, out_ref, d_ref, n_ref) = refs
        oh = _onehot_bounds(ec_ref[...], T, G)
        hseg = _gmm(oh, h_ref[...])
        xv = x_ref[...]
        s = jnp.sum(xv * hseg, axis=1, keepdims=True)
        mseg = jnp.sum(oh * m_ref[...], axis=1, keepdims=True)
        a = jnp.exp(s - mseg)
        both = _segsum(oh, jnp.concatenate([a, a * xv], axis=1))
        first = pl.program_id(0) == 0
        de = _acc(d_ref, both[:, 0:1], first)
        ne = _acc(n_ref, both[:, 1:33], first)
        if readout is not None:
            @pl.when(pl.program_id(0) == grid - 1)
            def _():
                rn = nn_ref[...] / (dnr[...] + 1e-12)
                re = ne / (de + 1e-12)
                z = jnp.concatenate(
                    [hn_ref[...], rn, h_ref[...], re, u_ref[...]], axis=1)
                z = _sp2(_mm(z, ow1r[...]) + ob1r[...])
                z = _sp2(_mm(z, ow2r[...]) + ob2r[...])
                xa = _mm(z, opwr[...]) + opbr[...]
                xl = _elu(_mm(latr[...], lw1r[...]) + lb1r[...])
                xl = _elu(_mm(xl, lw2r[...]) + lb2r[...])
                xl = _elu(_mm(xl, lw3r[...]) + lb3r[...])
                xp = _elu(_mm(fprer[...], pw1r[...]) + pb1r[...])
                xp = _elu(_mm(xp, pw2r[...]) + pb2r[...])
                xp = _elu(_mm(xp, pw3r[...]) + pb3r[...])
                ohg = _onehot(sgr[0], 256)
                xs = _gmm(ohg, embr[...])
                fx = jnp.concatenate([xa, xl, xp, xs], axis=1)
                out_ref[...] = _mm(fx, woutr[...]) + boutr[...]

    if readout is None:
        return pl.pallas_call(
            body, grid=(grid,),
            in_specs=[pl.BlockSpec((T, 32), lambda i: (i, 0)),
                      _full(cnt_row), _full(h), _full(m)],
            out_specs=[_cfix((G, 1)), _cfix((G, 32))],
            out_shape=[jax.ShapeDtypeStruct((G, 1), F32),
                       jax.ShapeDtypeStruct((G, 32), F32)],
        )(x, cnt_row, h, m)
    ro = readout
    sg3 = ro[24]
    return pl.pallas_call(
        body, grid=(grid,),
        in_specs=([pl.BlockSpec((T, 32), lambda i: (i, 0)),
                   _full(cnt_row), _full(h), _full(m)]
                  + [_full(a) for a in ro[:24]]
                  + [pl.BlockSpec(sg3.shape, lambda i: (0, 0, 0))]
                  + [_full(a) for a in ro[25:]]),
        out_specs=[_cfix((G, 128)), _cfix((G, 1)), _cfix((G, 32))],
        out_shape=[jax.ShapeDtypeStruct((G, 128), F32),
                   jax.ShapeDtypeStruct((G, 1), F32),
                   jax.ShapeDtypeStruct((G, 32), F32)],
    )(x, cnt_row, h, m, *ro)[0]


def _sc_gather_pair(table, isrc, idst):
    E = isrc.shape[0]
    D = table.shape[1]
    epw = E // _NW
    ch = 400
    nch = epw // ch
    mesh = plsc.VectorSubcoreMesh(core_axis_name="c", subcore_axis_name="s")

    @functools.partial(
        pl.kernel,
        out_type=[
            jax.ShapeDtypeStruct((E, D), F32),
            jax.ShapeDtypeStruct((E, D), F32),
        ],
        mesh=mesh,
        compiler_params=pltpu.CompilerParams(use_tc_tiling_on_sc=False),
        scratch_types=[
            [pltpu.VMEM((ch,), jnp.int32)] * 2,
            [pltpu.VMEM((ch, D), F32)] * 2,
            [pltpu.VMEM((ch,), jnp.int32)] * 2,
            [pltpu.VMEM((ch, D), F32)] * 2,
            [pltpu.SemaphoreType.DMA] * 2,
            [pltpu.SemaphoreType.DMA] * 2,
        ],
    )
    def k(table_h, isrc_h, idst_h, osrc_h, odst_h, iv1, rv1, iv2, rv2, s1, s2):
        wid = lax.axis_index("s") * _NC + lax.axis_index("c")
        base = wid * epw

        def load_and_fire(c):
            b = c % 2
            off = base + c * ch
            pltpu.sync_copy(isrc_h.at[pl.ds(off, ch)], iv1[b])
            pltpu.sync_copy(idst_h.at[pl.ds(off, ch)], iv2[b])
            g1 = pltpu.async_copy(table_h.at[iv1[b]], rv1[b], s1[b])
            g2 = pltpu.async_copy(table_h.at[iv2[b]], rv2[b], s2[b])
            return g1, g2

        def drain(c, g1, g2):
            b = c % 2
            off = base + c * ch
            g1.wait()
            g2.wait()
            pltpu.sync_copy(rv1[b], osrc_h.at[pl.ds(off, ch)])
            pltpu.sync_copy(rv2[b], odst_h.at[pl.ds(off, ch)])

        pend = load_and_fire(0)
        for c in range(1, nch):
            nxt = load_and_fire(c)
            drain(c - 1, *pend)
            pend = nxt
        drain(nch - 1, *pend)

    return k(table, isrc, idst)


def _sc_scatter_add(data, idx, gidx, nrows, G, zeros):
    E = idx.shape[0]
    D = data.shape[1]
    epw = E // _NW
    ch = 2000
    nch = epw // ch
    stripe = nrows // _NS
    gstripe = G // _NS
    mesh = plsc.VectorSubcoreMesh(core_axis_name="c", subcore_axis_name="s")

    @functools.partial(
        pl.kernel,
        out_type=[jax.ShapeDtypeStruct((_NC, nrows, D), F32),
                  jax.ShapeDtypeStruct((_NC, G, D), F32)],
        mesh=mesh,
        compiler_params=pltpu.CompilerParams(use_tc_tiling_on_sc=False),
        scratch_types=[
            pltpu.VMEM((ch,), jnp.int32),
            pltpu.VMEM((ch,), jnp.int32),
            pltpu.VMEM((ch, D), F32),
            pltpu.VMEM_SHARED((nrows, D), F32),
            pltpu.VMEM_SHARED((G, D), F32),
        ],
    )
    def k(data_h, idx_h, gidx_h, zeros_h, out_h, gout_h, iv, gv, dv, acc, gacc):
        cid = lax.axis_index("c")
        sid = lax.axis_index("s")
        wid = sid * _NC + cid
        pltpu.sync_copy(zeros_h.at[pl.ds(sid * stripe, stripe)],
                        acc.at[pl.ds(sid * stripe, stripe)])
        pltpu.sync_copy(zeros_h.at[pl.ds(sid * gstripe, gstripe)],
                        gacc.at[pl.ds(sid * gstripe, gstripe)])
        plsc.subcore_barrier()
        base = wid * epw
        for c in range(nch):
            off = base + c * ch
            pltpu.sync_copy(idx_h.at[pl.ds(off, ch)], iv)
            pltpu.sync_copy(gidx_h.at[pl.ds(off, ch)], gv)
            pltpu.sync_copy(data_h.at[pl.ds(off, ch)], dv)
            pltpu.sync_copy(dv, acc.at[iv], add=True)
            pltpu.sync_copy(dv, gacc.at[gv], add=True)
        plsc.subcore_barrier()
        pltpu.sync_copy(acc.at[pl.ds(sid * stripe, stripe)],
                        out_h.at[cid, pl.ds(sid * stripe, stripe)])
        pltpu.sync_copy(gacc.at[pl.ds(sid * gstripe, gstripe)],
                        gout_h.at[cid, pl.ds(sid * gstripe, gstripe)])

    return k(data, idx, gidx, zeros)


def _sc_scatter_ones(idx, gidx, nrows, G, zeros, ones_ch):
    E = idx.shape[0]
    ch, D = ones_ch.shape
    epw = E // _NW
    nch = epw // ch
    stripe = nrows // _NS
    gstripe = G // _NS
    mesh = plsc.VectorSubcoreMesh(core_axis_name="c", subcore_axis_name="s")

    @functools.partial(
        pl.kernel,
        out_type=[jax.ShapeDtypeStruct((_NC, nrows, D), F32),
                  jax.ShapeDtypeStruct((_NC, G, D), F32)],
        mesh=mesh,
        compiler_params=pltpu.CompilerParams(use_tc_tiling_on_sc=False),
        scratch_types=[
            pltpu.VMEM((ch,), jnp.int32),
            pltpu.VMEM((ch,), jnp.int32),
            pltpu.VMEM((ch, D), F32),
            pltpu.VMEM_SHARED((nrows, D), F32),
            pltpu.VMEM_SHARED((G, D), F32),
        ],
    )
    def k(idx_h, gidx_h, zeros_h, ones_h, out_h, gout_h, iv, gv, dv, acc, gacc):
        cid = lax.axis_index("c")
        sid = lax.axis_index("s")
        wid = sid * _NC + cid
        pltpu.sync_copy(zeros_h.at[pl.ds(sid * stripe, stripe)],
                        acc.at[pl.ds(sid * stripe, stripe)])
        pltpu.sync_copy(zeros_h.at[pl.ds(sid * gstripe, gstripe)],
                        gacc.at[pl.ds(sid * gstripe, gstripe)])
        pltpu.sync_copy(ones_h, dv)
        plsc.subcore_barrier()
        base = wid * epw
        for c in range(nch):
            off = base + c * ch
            pltpu.sync_copy(idx_h.at[pl.ds(off, ch)], iv)
            pltpu.sync_copy(gidx_h.at[pl.ds(off, ch)], gv)
            pltpu.sync_copy(dv, acc.at[iv], add=True)
            pltpu.sync_copy(dv, gacc.at[gv], add=True)
        plsc.subcore_barrier()
        pltpu.sync_copy(acc.at[pl.ds(sid * stripe, stripe)],
                        out_h.at[cid, pl.ds(sid * stripe, stripe)])
        pltpu.sync_copy(gacc.at[pl.ds(sid * gstripe, gstripe)],
                        gout_h.at[cid, pl.ds(sid * gstripe, gstripe)])

    return k(idx, gidx, zeros, ones_ch)


def kernel(edge_index, bond_dist, node_feat, state_attr, node_graph_idx,
           edge_graph_idx, sg, lattice, fpretrain, params):
    E = bond_dist.shape[0]
    N = node_feat.shape[0]
    G = state_attr.shape[0]

    src = edge_index[0].astype(jnp.int32)
    dst = edge_index[1].astype(jnp.int32)
    gi_n3 = node_graph_idx.astype(jnp.int32).reshape(N // _TN, _TN, 1)
    nf3 = node_feat.astype(jnp.int32).reshape(N // _TN, _TN, 1)
    sg3 = sg.astype(jnp.int32).reshape(1, G, 1)
    bd = bond_dist.reshape(E, 1)

    p = params
    rb = lambda b: b.reshape(1, -1)

    (ew1, eb1), (ew2, eb2) = p['edge_enc']
    enc = (jnp.pad(ew1, ((0, 28), (0, 0))), rb(eb1), ew2, rb(eb2))

    (nw1, nb1), (nw2, nb2) = p['node_enc']
    (sw1, sb1), (sw2, sb2) = p['state_enc']
    embp = jnp.pad(p['emb_atom'], ((0, 33), (0, 0)))
    v, u = _node_encoder(nf3, embp, nw1, rb(nb1), nw2, rb(nb2),
                         state_attr, sw1, rb(sb1), sw2, rb(sb2))

    zeros_n = jnp.zeros((N, 32), F32)
    gie = edge_graph_idx.astype(jnp.int32)
    counts, gcounts = _sc_scatter_ones(dst, gie, N, G, zeros_n,
                                       jnp.ones((2000, 32), F32))
    ecnt_row = (gcounts[0, :, 0] + gcounts[1, :, 0]).reshape(1, G)
    ecnt_col = ecnt_row.reshape(G, 1)

    e_in = bd
    vcnt_row = vcnt_col = None
    qe = None
    seg_v_post = None
    for bi, blk in enumerate(p['blocks']):
        (bw1, bb1), (bw2, bb2), (bw3, bb3) = blk['edge']
        vs, vd = _sc_gather_pair(v, src, dst)
        if bi == 0:
            em, en, q0 = _edge_block(
                vs, vd, e_in, u, bw1, rb(bb1), bw2, rb(bb2), bw3, rb(bb3),
                enc=enc, ecnt=ecnt_row)
            qe = q0
        else:
            em, en = _edge_block(
                vs, vd, e_in, u, bw1, rb(bb1), bw2, rb(bb2), bw3, rb(bb3),
                ecnt=ecnt_row)
        aggs, gsums = _sc_scatter_add(em, dst, gie, N, G, zeros_n)
        (vw1, vb1), (vw2, vb2), (vw3, vb3) = blk['node']
        (uw1, ub1), (uw2, ub2), (uw3, ub3) = blk['state']
        if bi == 0:
            vn, sv_post, un, qe, vcnt_col, vcnt_row, _sv_pre = _node_block(
                v, aggs, counts, u,
                vw1, rb(vb1), vw2, rb(vb2), vw3, rb(vb3),
                gsums, qe, ecnt_col,
                uw1, rb(ub1), uw2, rb(ub2), uw3, rb(ub3),
                gi3=gi_n3)
        else:
            vn, sv_post, un, qe, _sv_pre = _node_block(
                v, aggs, counts, u,
                vw1, rb(vb1), vw2, rb(vb2), vw3, rb(vb3),
                gsums, qe, ecnt_col,
                uw1, rb(ub1), uw2, rb(ub2), uw3, rb(ub3),
                vcnt_row=vcnt_row, vcnt_col=vcnt_col)
        e_in, v, u = en, vn, un
        seg_v_post = sv_post

    mn, hn, he = _s2s_pass_a(
        v, (seg_v_post, vcnt_col, qe, ecnt_col,
            p['s2s_node']['Wih'], rb(p['s2s_node']['b']),
            p['s2s_edge']['Wih'], rb(p['s2s_edge']['b'])), vcnt_row)
    me_ = _s2s_pass_a(e_in, he, ecnt_row)
    den_n, num_n = _s2s_pass_b(v, hn, vcnt_row, mn)

    (ow1, ob1), (ow2, ob2) = p['out_mlp']
    opw, opb = p['out_proj']
    (lw1, lb1), (lw2, lb2), (lw3, lb3) = p['emb_lattice']
    (pw1, pb1), (pw2, pb2), (pw3, pb3) = p['emb_pretrain']
    embsg = jnp.pad(p['emb_sg'], ((0, 26), (0, 0)))
    wout, bout = p['output_layer']

    return _s2s_pass_b(
        e_in, he, ecnt_row, me_,
        readout=(hn, num_n, den_n, u,
                 ow1, rb(ob1), ow2, rb(ob2), opw, rb(opb),
                 lattice, lw1, rb(lb1), lw2, rb(lb2), lw3, rb(lb3),
                 fpretrain, pw1, rb(pb1), pw2, rb(pb2), pw3, rb(pb3),
                 sg3, embsg, wout, rb(bout)))
